# Initial kernel scaffold; baseline (speedup 1.0000x reference)
import jax
import jax.numpy as jnp
from jax.experimental import pallas as pl


def kernel(x, codebook):
    raise NotImplementedError("write your pallas kernel here")


# speedup vs baseline: 1.1886x; 1.1886x over previous
import functools

import jax
import jax.numpy as jnp
from jax import lax
from jax.experimental import pallas as pl
from jax.experimental.pallas import tpu as pltpu
from jax.experimental.pallas import tpu_sc as plsc

_TB = 256
_KT = 4096


def _vq_argmin_body(flatt_ref, cbm2_ref, idx_ref, loss_ref, *, n_ktiles, k_total):
    blkt = flatt_ref[...]
    xx = jnp.sum(blkt * blkt, axis=0, keepdims=True)
    blkt_r = blkt.astype(jnp.bfloat16).astype(jnp.float32)
    best_m = jnp.full((_TB,), jnp.inf, dtype=jnp.float32)
    best_loss = jnp.zeros((_TB,), dtype=jnp.float32)
    best_i = jnp.zeros((_TB,), dtype=jnp.int32)
    for kt in range(n_ktiles):
        cb = cbm2_ref[kt * _KT:(kt + 1) * _KT, :]
        mm2 = jnp.dot(cb, blkt_r, preferred_element_type=jnp.float32)
        cc = 0.25 * jnp.sum(cb * cb, axis=1, keepdims=True)
        dist = (xx + mm2) + cc
        m = jnp.min(dist, axis=0)
        ii = lax.broadcasted_iota(jnp.int32, (_KT, _TB), 0) + (kt * _KT)
        cand = jnp.min(jnp.where(dist == m[None, :], ii, k_total), axis=0)
        upd = m < best_m
        best_i = jnp.where(upd, cand, best_i)
        best_loss = jnp.where(upd, m, best_loss)
        best_m = jnp.where(upd, m.astype(jnp.bfloat16).astype(jnp.float32),
                           best_m)
    idx_ref[0, 0, :] = best_i
    part = jnp.sum(best_loss).reshape(1, 1)

    @pl.when(pl.program_id(0) == 0)
    def _init():
        loss_ref[...] = jnp.zeros_like(part)

    loss_ref[...] += part


def _compute_indices(flatt, cbm2, *, interpret=False):
    d, n = flatt.shape
    k_total = cbm2.shape[0]
    nb = n // _TB
    body = functools.partial(_vq_argmin_body, n_ktiles=k_total // _KT,
                             k_total=k_total)
    idx3, loss = pl.pallas_call(
        body,
        grid=(nb,),
        in_specs=[
            pl.BlockSpec((d, _TB), lambda i: (0, i)),
            pl.BlockSpec((k_total, d), lambda i: (0, 0)),
        ],
        out_specs=[
            pl.BlockSpec((1, 1, _TB), lambda i: (i, 0, 0)),
            pl.BlockSpec((1, 1), lambda i: (0, 0)),
        ],
        out_shape=[
            jax.ShapeDtypeStruct((nb, 1, _TB), jnp.int32),
            jax.ShapeDtypeStruct((1, 1), jnp.float32),
        ],
        interpret=interpret,
    )(flatt, cbm2)
    return idx3.reshape(n), loss[0, 0]


def _make_sc_gather(n, d):
    info = plsc.get_sparse_core_info()
    nw = info.num_cores * info.num_subcores
    assert n % (8 * nw) == 0 and d % info.num_lanes == 0
    b_per_w = n // nw
    mesh = plsc.VectorSubcoreMesh(core_axis_name="c", subcore_axis_name="s")

    @functools.partial(
        pl.kernel, mesh=mesh,
        compiler_params=pltpu.CompilerParams(use_tc_tiling_on_sc=False),
        out_type=jax.ShapeDtypeStruct((n, d), jnp.float32),
        scratch_types=[
            pltpu.VMEM((b_per_w,), jnp.int32),
            pltpu.VMEM((b_per_w, d), jnp.float32),
            pltpu.SemaphoreType.DMA,
        ],
    )
    def gather_k(idx_hbm, table_hbm, out_hbm, idx_v, rows_v, sem):
        wid = lax.axis_index("s") * info.num_cores + lax.axis_index("c")
        base = wid * b_per_w
        pltpu.sync_copy(idx_hbm.at[pl.ds(base, b_per_w)], idx_v)
        pltpu.async_copy(table_hbm.at[idx_v], rows_v, sem).wait()
        pltpu.sync_copy(rows_v, out_hbm.at[pl.ds(base, b_per_w)])

    return gather_k


def kernel(x, codebook):
    b, d, t = x.shape
    k_total = codebook.shape[0]
    n = b * t
    flatt = jnp.transpose(x, (1, 0, 2)).reshape(d, n)
    cbm2 = -2.0 * codebook

    indices, loss_sum = _compute_indices(flatt, cbm2)
    quant = _make_sc_gather(n, d)(indices, codebook)

    out = jnp.transpose(quant.reshape(b, t, d), (0, 2, 1))
    commit_loss = loss_sum / jnp.float32(n * d)
    return out, indices.reshape(b, t), commit_loss


# speedup vs baseline: 1.1916x; 1.0025x over previous
import functools

import jax
import jax.numpy as jnp
from jax import lax
from jax.experimental import pallas as pl
from jax.experimental.pallas import tpu as pltpu
from jax.experimental.pallas import tpu_sc as plsc

_TB = 256
_KT = 4096


def _vq_argmin_body(x_ref, cbm2_ref, idx_ref, loss_ref, *, n_ktiles, k_total):
    blkt = x_ref[0]
    xx = jnp.sum(blkt * blkt, axis=0, keepdims=True)
    blkt_r = blkt.astype(jnp.bfloat16).astype(jnp.float32)
    best_m = jnp.full((_TB,), jnp.inf, dtype=jnp.float32)
    best_loss = jnp.zeros((_TB,), dtype=jnp.float32)
    best_i = jnp.zeros((_TB,), dtype=jnp.int32)
    for kt in range(n_ktiles):
        cb = cbm2_ref[kt * _KT:(kt + 1) * _KT, :]
        mm2 = jnp.dot(cb, blkt_r, preferred_element_type=jnp.float32)
        cc = 0.25 * jnp.sum(cb * cb, axis=1, keepdims=True)
        dist = (xx + mm2) + cc
        m = jnp.min(dist, axis=0)
        ii = lax.broadcasted_iota(jnp.int32, (_KT, _TB), 0) + (kt * _KT)
        cand = jnp.min(jnp.where(dist == m[None, :], ii, k_total), axis=0)
        upd = m < best_m
        best_i = jnp.where(upd, cand, best_i)
        best_loss = jnp.where(upd, m, best_loss)
        best_m = jnp.where(upd, m.astype(jnp.bfloat16).astype(jnp.float32),
                           best_m)
    idx_ref[0, 0, :] = best_i
    part = jnp.sum(best_loss).reshape(1, 1)

    @pl.when(pl.program_id(0) == 0)
    def _init():
        loss_ref[...] = jnp.zeros_like(part)

    loss_ref[...] += part


def _compute_indices(x, cbm2, *, interpret=False):
    b, d, t = x.shape
    n = b * t
    k_total = cbm2.shape[0]
    nb = n // _TB
    tpb = t // _TB
    body = functools.partial(_vq_argmin_body, n_ktiles=k_total // _KT,
                             k_total=k_total)
    idx3, loss = pl.pallas_call(
        body,
        grid=(nb,),
        in_specs=[
            pl.BlockSpec((1, d, _TB), lambda i: (i // tpb, 0, i % tpb)),
            pl.BlockSpec((k_total, d), lambda i: (0, 0)),
        ],
        out_specs=[
            pl.BlockSpec((1, 1, _TB), lambda i: (i, 0, 0)),
            pl.BlockSpec((1, 1), lambda i: (0, 0)),
        ],
        out_shape=[
            jax.ShapeDtypeStruct((nb, 1, _TB), jnp.int32),
            jax.ShapeDtypeStruct((1, 1), jnp.float32),
        ],
        interpret=interpret,
    )(x, cbm2)
    return idx3.reshape(n), loss[0, 0]


def _make_sc_gather(n, d):
    info = plsc.get_sparse_core_info()
    nw = info.num_cores * info.num_subcores
    assert n % (8 * nw) == 0 and d % info.num_lanes == 0
    b_per_w = n // nw
    mesh = plsc.VectorSubcoreMesh(core_axis_name="c", subcore_axis_name="s")

    @functools.partial(
        pl.kernel, mesh=mesh,
        compiler_params=pltpu.CompilerParams(use_tc_tiling_on_sc=False),
        out_type=jax.ShapeDtypeStruct((n, d), jnp.float32),
        scratch_types=[
            pltpu.VMEM((b_per_w,), jnp.int32),
            pltpu.VMEM((b_per_w, d), jnp.float32),
            pltpu.SemaphoreType.DMA,
        ],
    )
    def gather_k(idx_hbm, table_hbm, out_hbm, idx_v, rows_v, sem):
        wid = lax.axis_index("s") * info.num_cores + lax.axis_index("c")
        base = wid * b_per_w
        pltpu.sync_copy(idx_hbm.at[pl.ds(base, b_per_w)], idx_v)
        pltpu.async_copy(table_hbm.at[idx_v], rows_v, sem).wait()
        pltpu.sync_copy(rows_v, out_hbm.at[pl.ds(base, b_per_w)])

    return gather_k


def kernel(x, codebook):
    b, d, t = x.shape
    k_total = codebook.shape[0]
    n = b * t
    cbm2 = -2.0 * codebook

    indices, loss_sum = _compute_indices(x, cbm2)
    quant = _make_sc_gather(n, d)(indices, codebook)

    out = jnp.transpose(quant.reshape(b, t, d), (0, 2, 1))
    commit_loss = loss_sum / jnp.float32(n * d)
    return out, indices.reshape(b, t), commit_loss


# speedup vs baseline: 1.3025x; 1.0930x over previous
import functools

import jax
import jax.numpy as jnp
from jax import lax
from jax.experimental import pallas as pl
from jax.experimental.pallas import tpu as pltpu
from jax.experimental.pallas import tpu_sc as plsc

_TB = 256
_KT = 4096


def _vq_argmin_body(x_ref, cbm2_ref, idx_ref, loss_ref, cc_scr, ii_scr, *,
                    n_ktiles, k_total):
    @pl.when(pl.program_id(0) == 0)
    def _prep():
        for kt in range(n_ktiles):
            cb = cbm2_ref[kt * _KT:(kt + 1) * _KT, :]
            ccv = 0.25 * jnp.sum(cb * cb, axis=1, keepdims=True)
            cc_scr[kt * _KT:(kt + 1) * _KT, :] = jnp.broadcast_to(
                ccv, (_KT, _TB))
        ii_scr[...] = lax.broadcasted_iota(jnp.int32, (_KT, _TB), 0)

    blkt = x_ref[0]
    xx = jnp.sum(blkt * blkt, axis=0, keepdims=True)
    blkt_r = blkt.astype(jnp.bfloat16).astype(jnp.float32)
    best_m = jnp.full((_TB,), jnp.inf, dtype=jnp.float32)
    best_loss = jnp.zeros((_TB,), dtype=jnp.float32)
    best_i = jnp.zeros((_TB,), dtype=jnp.int32)
    for kt in range(n_ktiles):
        cb = cbm2_ref[kt * _KT:(kt + 1) * _KT, :]
        mm2 = jnp.dot(cb, blkt_r, preferred_element_type=jnp.float32)
        dist = (xx + mm2) + cc_scr[kt * _KT:(kt + 1) * _KT, :]
        m = jnp.min(dist, axis=0)
        cand = jnp.min(jnp.where(dist == m[None, :], ii_scr[...], k_total),
                       axis=0)
        upd = m < best_m
        best_i = jnp.where(upd, cand + (kt * _KT), best_i)
        best_loss = jnp.where(upd, m, best_loss)
        best_m = jnp.where(upd, m.astype(jnp.bfloat16).astype(jnp.float32),
                           best_m)
    idx_ref[0, 0, :] = best_i
    part = jnp.sum(best_loss).reshape(1, 1)

    @pl.when(pl.program_id(0) == 0)
    def _init():
        loss_ref[...] = jnp.zeros_like(part)

    loss_ref[...] += part


def _compute_indices(x, cbm2, *, interpret=False):
    b, d, t = x.shape
    n = b * t
    k_total = cbm2.shape[0]
    nb = n // _TB
    tpb = t // _TB
    body = functools.partial(_vq_argmin_body, n_ktiles=k_total // _KT,
                             k_total=k_total)
    idx3, loss = pl.pallas_call(
        body,
        grid=(nb,),
        in_specs=[
            pl.BlockSpec((1, d, _TB), lambda i: (i // tpb, 0, i % tpb)),
            pl.BlockSpec((k_total, d), lambda i: (0, 0)),
        ],
        out_specs=[
            pl.BlockSpec((1, 1, _TB), lambda i: (i, 0, 0)),
            pl.BlockSpec((1, 1), lambda i: (0, 0)),
        ],
        out_shape=[
            jax.ShapeDtypeStruct((nb, 1, _TB), jnp.int32),
            jax.ShapeDtypeStruct((1, 1), jnp.float32),
        ],
        scratch_shapes=[
            pltpu.VMEM((k_total, _TB), jnp.float32),
            pltpu.VMEM((_KT, _TB), jnp.int32),
        ],
        interpret=interpret,
    )(x, cbm2)
    return idx3.reshape(n), loss[0, 0]


def _make_sc_gather(n, d):
    info = plsc.get_sparse_core_info()
    nw = info.num_cores * info.num_subcores
    assert n % (8 * nw) == 0 and d % info.num_lanes == 0
    b_per_w = n // nw
    mesh = plsc.VectorSubcoreMesh(core_axis_name="c", subcore_axis_name="s")

    @functools.partial(
        pl.kernel, mesh=mesh,
        compiler_params=pltpu.CompilerParams(use_tc_tiling_on_sc=False),
        out_type=jax.ShapeDtypeStruct((n, d), jnp.float32),
        scratch_types=[
            pltpu.VMEM((b_per_w,), jnp.int32),
            pltpu.VMEM((b_per_w, d), jnp.float32),
            pltpu.SemaphoreType.DMA,
        ],
    )
    def gather_k(idx_hbm, table_hbm, out_hbm, idx_v, rows_v, sem):
        wid = lax.axis_index("s") * info.num_cores + lax.axis_index("c")
        base = wid * b_per_w
        pltpu.sync_copy(idx_hbm.at[pl.ds(base, b_per_w)], idx_v)
        pltpu.async_copy(table_hbm.at[idx_v], rows_v, sem).wait()
        pltpu.sync_copy(rows_v, out_hbm.at[pl.ds(base, b_per_w)])

    return gather_k


def kernel(x, codebook):
    b, d, t = x.shape
    k_total = codebook.shape[0]
    n = b * t
    cbm2 = -2.0 * codebook

    indices, loss_sum = _compute_indices(x, cbm2)
    quant = _make_sc_gather(n, d)(indices, codebook)

    out = jnp.transpose(quant.reshape(b, t, d), (0, 2, 1))
    commit_loss = loss_sum / jnp.float32(n * d)
    return out, indices.reshape(b, t), commit_loss
